# baseline (speedup 1.0000x reference)
import functools

import jax
import jax.numpy as jnp
import numpy as np
from jax import lax
from jax.experimental import pallas as pl
from jax.experimental.pallas import tpu as pltpu
from jax.experimental.pallas import tpu_sc as plsc

VOCAB = 25
EMBED = 128
PHYS = 64
SEQ = 100
BATCH = 4096

VSLOT = 32
NC, NS = 2, 16
NW = NC * NS
KBUF = 4
SEQ_PER_W = BATCH // NW
SEQ_PAD = 104


def _positional_encoding(max_len, d_model):
    position = np.arange(max_len)[:, None].astype(np.float64)
    div_term = np.exp(np.arange(0, d_model, 2).astype(np.float64)
                      * (-np.log(10000.0) / d_model))
    pe = np.zeros((max_len, d_model), dtype=np.float32)
    pe[:, 0::2] = np.sin(position * div_term)
    pe[:, 1::2] = np.cos(position * div_term)
    return pe


SEQ_TAB = 104
LSLOTS = 5

_PE = np.zeros((SEQ_TAB, EMBED), dtype=np.float32)
_PE[:SEQ] = _positional_encoding(SEQ, EMBED)


def _table_body(tok_ref, phys_ref, w_ref, b_ref, pe_ref, g_ref, beta_ref, out_ref):
    vmask = (lax.broadcasted_iota(jnp.int32, (VOCAB, 1), 0) != 0).astype(jnp.float32)
    tok = tok_ref[...] * vmask
    phys = phys_ref[...] * vmask
    w = w_ref[...]
    w_tok = w[:, :EMBED]
    w_phys = w[:, EMBED:]
    dn = (((1,), (1,)), ((), ()))
    tv = (lax.dot_general(tok, w_tok, dn, preferred_element_type=jnp.float32)
          + lax.dot_general(phys, w_phys, dn, preferred_element_type=jnp.float32))
    pos = (lax.dot_general(pe_ref[...], w_tok, dn, preferred_element_type=jnp.float32)
           + b_ref[...][None, :])
    tv32 = jnp.concatenate([tv, jnp.zeros((VSLOT - VOCAB, EMBED), jnp.float32)], 0)
    fused = pos[:, None, :] + tv32[None, :, :]
    mean = jnp.mean(fused, axis=-1, keepdims=True)
    var = jnp.mean((fused - mean) ** 2, axis=-1, keepdims=True)
    out_ref[...] = ((fused - mean) * lax.rsqrt(var + 1e-5)
                    * g_ref[...][None, None, :] + beta_ref[...][None, None, :])


def _build_table(tok_table, phys_table, fusion_W, fusion_b, pe, ln_gamma, ln_beta):
    return pl.pallas_call(
        _table_body,
        out_shape=jax.ShapeDtypeStruct((SEQ_TAB, VSLOT, EMBED), jnp.float32),
    )(tok_table, phys_table, fusion_W, fusion_b, pe, ln_gamma, ln_beta)


CHUNK = 128
N_CHUNKS = BATCH * SEQ // NW // CHUNK


def _gather_body(table_hbm, x_hbm, out_hbm, xv, idxv, tslice, rows, *sems):
    gsems, wsems = sems[:KBUF], sems[KBUF:]
    wid = lax.axis_index("s") * NC + lax.axis_index("c")
    n_elems = SEQ_PER_W * SEQ
    base = wid * n_elems
    pltpu.sync_copy(x_hbm.at[pl.ds(base, n_elems)], xv)
    @pl.when(lax.axis_index("s") == 0)
    def _():
        pltpu.sync_copy(table_hbm, tslice)

    def compute_idx(c):
        l_c = (base + c * CHUNK) // BATCH
        for k in range(CHUNK // 16):
            idxv[c, pl.ds(k * 16, 16)] = (xv[pl.ds(c * CHUNK + k * 16, 16)]
                                          + l_c * VSLOT)

    for b in range(KBUF):
        compute_idx(b)
    plsc.subcore_barrier()

    def issue_gather(c, b):
        pltpu.async_copy(tslice.at[idxv.at[c]], rows.at[b], gsems[b])

    def out_slice(c):
        return out_hbm.at[pl.ds(base + c * CHUNK, CHUNK)]

    for b in range(KBUF):
        issue_gather(b, b)

    def round_body(g, _):
        for b in range(KBUF):
            c = g * KBUF + b
            nxt = c + KBUF

            @pl.when(nxt < N_CHUNKS)
            def _():
                compute_idx(nxt)

            pltpu.make_async_copy(tslice.at[idxv.at[0]],
                                  rows.at[b], gsems[b]).wait()
            pltpu.async_copy(rows.at[b], out_slice(c), wsems[b])
        for b in range(KBUF):
            c = g * KBUF + b
            pltpu.make_async_copy(rows.at[b], out_slice(c), wsems[b]).wait()
            nxt = c + KBUF

            @pl.when(nxt < N_CHUNKS)
            def _():
                issue_gather(nxt, b)
        return 0

    lax.fori_loop(0, N_CHUNKS // KBUF, round_body, 0)


@jax.jit
def kernel(x, tok_table, phys_table, fusion_W, fusion_b, ln_gamma, ln_beta):
    pe = jnp.asarray(_PE)
    table = _build_table(tok_table, phys_table, fusion_W, fusion_b,
                         pe, ln_gamma, ln_beta)
    table2d = table.reshape(SEQ_TAB * VSLOT, EMBED)

    mesh = plsc.VectorSubcoreMesh(core_axis_name="c", subcore_axis_name="s")
    gather = functools.partial(
        pl.kernel,
        mesh=mesh,
        compiler_params=pltpu.CompilerParams(use_tc_tiling_on_sc=True),
        out_type=jax.ShapeDtypeStruct((SEQ * BATCH, EMBED), jnp.float32),
        scratch_types=[
            pltpu.VMEM((N_CHUNKS * CHUNK,), jnp.int32),
            pltpu.VMEM((N_CHUNKS, CHUNK), jnp.int32),
            pltpu.VMEM_SHARED((SEQ_TAB * VSLOT, EMBED), jnp.float32),
            pltpu.VMEM((KBUF, CHUNK, EMBED), jnp.float32),
        ] + [pltpu.SemaphoreType.DMA] * (2 * KBUF),
    )(_gather_body)
    out = gather(table2d, x.astype(jnp.int32).T.reshape(-1))
    return out.reshape(SEQ, BATCH, EMBED).transpose(1, 0, 2)


# speedup vs baseline: 1.0110x; 1.0110x over previous
import functools

import jax
import jax.numpy as jnp
import numpy as np
from jax import lax
from jax.experimental import pallas as pl
from jax.experimental.pallas import tpu as pltpu
from jax.experimental.pallas import tpu_sc as plsc

VOCAB = 25
EMBED = 128
PHYS = 64
SEQ = 100
BATCH = 4096

VSLOT = 32
NC, NS = 2, 16
NW = NC * NS
KBUF = 4
SEQ_PER_W = BATCH // NW
SEQ_PAD = 104


def _positional_encoding(max_len, d_model):
    position = np.arange(max_len)[:, None].astype(np.float64)
    div_term = np.exp(np.arange(0, d_model, 2).astype(np.float64)
                      * (-np.log(10000.0) / d_model))
    pe = np.zeros((max_len, d_model), dtype=np.float32)
    pe[:, 0::2] = np.sin(position * div_term)
    pe[:, 1::2] = np.cos(position * div_term)
    return pe


SEQ_TAB = 104
LSLOTS = 5

_PE = np.zeros((SEQ_TAB, EMBED), dtype=np.float32)
_PE[:SEQ] = _positional_encoding(SEQ, EMBED)


def _table_body(tok_ref, phys_ref, w_ref, b_ref, pe_ref, g_ref, beta_ref, out_ref):
    vmask = (lax.broadcasted_iota(jnp.int32, (VOCAB, 1), 0) != 0).astype(jnp.float32)
    tok = tok_ref[...] * vmask
    phys = phys_ref[...] * vmask
    w = w_ref[...]
    w_tok = w[:, :EMBED]
    w_phys = w[:, EMBED:]
    dn = (((1,), (1,)), ((), ()))
    tv = (lax.dot_general(tok, w_tok, dn, preferred_element_type=jnp.float32)
          + lax.dot_general(phys, w_phys, dn, preferred_element_type=jnp.float32))
    pos = (lax.dot_general(pe_ref[...], w_tok, dn, preferred_element_type=jnp.float32)
           + b_ref[...][None, :])
    tv32 = jnp.concatenate([tv, jnp.zeros((VSLOT - VOCAB, EMBED), jnp.float32)], 0)
    fused = pos[:, None, :] + tv32[None, :, :]
    mean = jnp.mean(fused, axis=-1, keepdims=True)
    var = jnp.mean((fused - mean) ** 2, axis=-1, keepdims=True)
    out_ref[...] = ((fused - mean) * lax.rsqrt(var + 1e-5)
                    * g_ref[...][None, None, :] + beta_ref[...][None, None, :])


def _build_table(tok_table, phys_table, fusion_W, fusion_b, pe, ln_gamma, ln_beta):
    return pl.pallas_call(
        _table_body,
        out_shape=jax.ShapeDtypeStruct((SEQ_TAB, VSLOT, EMBED), jnp.float32),
    )(tok_table, phys_table, fusion_W, fusion_b, pe, ln_gamma, ln_beta)


CHUNK = 128
N_CHUNKS = BATCH * SEQ // NW // CHUNK


def _gather_body(table_hbm, x_hbm, out_hbm, xv, idxv, tslice, rows, *sems):
    gsems, wsems = sems[:KBUF], sems[KBUF:]
    wid = lax.axis_index("s") * NC + lax.axis_index("c")
    n_elems = SEQ_PER_W * SEQ
    base = wid * n_elems
    pltpu.sync_copy(x_hbm.at[pl.ds(base, n_elems)], xv)
    @pl.when(lax.axis_index("s") == 0)
    def _():
        pltpu.sync_copy(table_hbm, tslice)

    def idx_body(c, _):
        l_c = (base + c * CHUNK) // BATCH
        for k in range(CHUNK // 16):
            idxv[c, pl.ds(k * 16, 16)] = (xv[pl.ds(c * CHUNK + k * 16, 16)]
                                          + l_c * VSLOT)
        return 0

    lax.fori_loop(0, N_CHUNKS, idx_body, 0)
    plsc.subcore_barrier()

    def issue_gather(c, b):
        pltpu.async_copy(tslice.at[idxv.at[c]], rows.at[b], gsems[b])

    def out_slice(c):
        return out_hbm.at[pl.ds(base + c * CHUNK, CHUNK)]

    for b in range(KBUF):
        issue_gather(b, b)

    def round_body(g, _):
        for b in range(KBUF):
            c = g * KBUF + b
            pltpu.make_async_copy(tslice.at[idxv.at[0]],
                                  rows.at[b], gsems[b]).wait()
            pltpu.async_copy(rows.at[b], out_slice(c), wsems[b])
        for b in range(KBUF):
            c = g * KBUF + b
            pltpu.make_async_copy(rows.at[b], out_slice(c), wsems[b]).wait()
            nxt = c + KBUF

            @pl.when(nxt < N_CHUNKS)
            def _():
                issue_gather(nxt, b)
        return 0

    lax.fori_loop(0, N_CHUNKS // KBUF, round_body, 0)


@jax.jit
def kernel(x, tok_table, phys_table, fusion_W, fusion_b, ln_gamma, ln_beta):
    pe = jnp.asarray(_PE)
    table = _build_table(tok_table, phys_table, fusion_W, fusion_b,
                         pe, ln_gamma, ln_beta)
    table2d = table.reshape(SEQ_TAB * VSLOT, EMBED)

    mesh = plsc.VectorSubcoreMesh(core_axis_name="c", subcore_axis_name="s")
    gather = functools.partial(
        pl.kernel,
        mesh=mesh,
        compiler_params=pltpu.CompilerParams(use_tc_tiling_on_sc=True),
        out_type=jax.ShapeDtypeStruct((SEQ * BATCH, EMBED), jnp.float32),
        scratch_types=[
            pltpu.VMEM((N_CHUNKS * CHUNK,), jnp.int32),
            pltpu.VMEM((N_CHUNKS, CHUNK), jnp.int32),
            pltpu.VMEM_SHARED((SEQ_TAB * VSLOT, EMBED), jnp.float32),
            pltpu.VMEM((KBUF, CHUNK, EMBED), jnp.float32),
        ] + [pltpu.SemaphoreType.DMA] * (2 * KBUF),
    )(_gather_body)
    out = gather(table2d, x.astype(jnp.int32).T.reshape(-1))
    return out.reshape(SEQ, BATCH, EMBED).transpose(1, 0, 2)
